# baseline (speedup 1.0000x reference)
import numpy as np
import jax
import jax.numpy as jnp
from jax import lax
from jax.experimental import pallas as pl
from jax.experimental.pallas import tpu as pltpu
from jax.experimental.pallas import tpu_sc as plsc

_NUM_LVL = 16
_MAX_RES = 2048
_MIN_RES = 16
_MAX_ENTRY = 2 ** 19
_MASK = _MAX_ENTRY - 1
_P0 = np.int32(3367900313 - (1 << 32))
_P1 = np.int32(2654435761 - (1 << 32))
_N_POINTS = 1048576

_factor_b = np.exp((np.log(_MAX_RES) - np.log(_MIN_RES)) / (_NUM_LVL - 1))
_RESOLUTIONS = [float(np.floor(_MIN_RES * _factor_b ** i)) for i in range(_NUM_LVL)]
_TABLE_SIZES = [int(min(r ** 2, _MAX_ENTRY)) for r in _RESOLUTIONS]

_NUM_LOCAL = 8
_NUM_REMOTE = _NUM_LVL - _NUM_LOCAL
_NSLOT = 3
_C = 128
_NW = 32
_PPW = _N_POINTS // _NW
_NCHUNK = _PPW // _C
_G = _C // 16


def _i32(v):
    return v if v.dtype == jnp.int32 else v.astype(jnp.int32)


def _loop32(n, unroll=None):
    def decorator(body):
        def fbody(i, c):
            body(c)
            return c + np.int32(1)
        lax.fori_loop(0, n, fbody, np.int32(0), unroll=unroll)
    return decorator


def _coords(x0, x1, res):
    rf = np.float32(res - 1.0)
    hi = np.float32(res - 1.0001)
    c0 = jnp.minimum(jnp.maximum(x0 * rf, np.float32(0.0)), hi)
    c1 = jnp.minimum(jnp.maximum(x1 * rf, np.float32(0.0)), hi)
    f0 = c0.astype(jnp.int32)
    f1 = c1.astype(jnp.int32)
    d0 = c0 - f0.astype(jnp.float32)
    d1 = c1 - f1.astype(jnp.float32)
    return f0, f1, d0, d1


def _corner_indices(lvl, f0, f1):
    if _TABLE_SIZES[lvl] == _MAX_ENTRY:
        a = f0 * _P0
        ap = a + _P0
        b = f1 * _P1
        bp = b + _P1
        i00 = (a ^ b) & _MASK
        i10 = (ap ^ b) & _MASK
        i01 = (a ^ bp) & _MASK
        i11 = (ap ^ bp) & _MASK
    else:
        r = np.int32(_RESOLUTIONS[lvl])
        i00 = f0 + f1 * r
        i10 = i00 + np.int32(1)
        i01 = i00 + r
        i11 = i01 + np.int32(1)
    return i00, i10, i01, i11


def _weights(d0, d1):
    mx = np.float32(1.0) - d0
    my = np.float32(1.0) - d1
    return mx * my, d0 * my, mx * d1, d0 * d1


def _body(*refs):
    xf_ref = refs[0]
    tloc_refs = refs[1:1 + _NUM_LOCAL]
    trem_refs = refs[1 + _NUM_LOCAL:1 + _NUM_LVL]
    out_ref = refs[1 + _NUM_LVL]
    s = 2 + _NUM_LVL
    xc = refs[s]
    loc = refs[s + 1:s + 1 + _NUM_LOCAL]
    idxb = refs[s + 1 + _NUM_LOCAL]
    rows = refs[s + 2 + _NUM_LOCAL]
    dbuf = refs[s + 3 + _NUM_LOCAL]
    qbuf = refs[s + 4 + _NUM_LOCAL]
    outc = refs[s + 5 + _NUM_LOCAL]
    sem = refs[s + 6 + _NUM_LOCAL]
    sem2 = refs[s + 7 + _NUM_LOCAL]
    sem_x = refs[s + 8 + _NUM_LOCAL]
    sem_o = refs[s + 9 + _NUM_LOCAL]

    iota = lax.iota(jnp.int32, 16)

    for l in range(_NUM_LOCAL):
        pltpu.async_copy(tloc_refs[l], loc[l], sem2).wait()

    wid = (_i32(lax.axis_index("s")) * jnp.int32(2)
           + _i32(lax.axis_index("c")))

    wbase = wid * jnp.int32(_PPW)
    lastbase = wbase + jnp.int32(_PPW - _C)

    pltpu.async_copy(xf_ref.at[pl.ds(wbase * jnp.int32(2), 2 * _C)],
                     xc.at[pl.ds(0, 2 * _C)], sem_x)
    pltpu.async_copy(outc.at[pl.ds(np.int32(_C), _C)],
                     out_ref.at[pl.ds(wbase, _C)], sem_o)

    @_loop32(_NCHUNK)
    def chunk_body(ci):
        base = wbase + ci * jnp.int32(_C)
        p = ci & jnp.int32(1)
        poff = p * jnp.int32(2 * _C)
        prow = p * jnp.int32(_C)

        pltpu.make_async_copy(
            xf_ref.at[pl.ds(base * jnp.int32(2), 2 * _C)],
            xc.at[pl.ds(poff, 2 * _C)], sem_x).wait()
        nbase = jnp.minimum(base + jnp.int32(_C), lastbase)
        pltpu.async_copy(
            xf_ref.at[pl.ds(nbase * jnp.int32(2), 2 * _C)],
            xc.at[pl.ds((jnp.int32(2 * _C)) - poff + jnp.int32(0) * nbase, 2 * _C)],
            sem_x)

        def point_group(g):
            row = g * jnp.int32(16) + iota
            rr = row + row + poff
            x0 = plsc.load_gather(xc, [rr])
            x1 = plsc.load_gather(xc, [rr + jnp.int32(1)])
            return row, x0, x1

        def fire_level(lvl, slot):
            li = lvl - _NUM_LOCAL

            @_loop32(_G, unroll=2)
            def bbody(g, lvl=lvl, slot=slot):
                row, x0, x1 = point_group(g)
                f0, f1, d0, d1 = _coords(x0, x1, _RESOLUTIONS[lvl])
                i00, i10, i01, i11 = _corner_indices(lvl, f0, f1)
                sl = pl.ds(g * jnp.int32(16), 16)
                two = jnp.int32(2)
                three = jnp.int32(3)
                idxb[np.int32(slot), np.int32(0), sl] = lax.shift_right_logical(i00, two)
                idxb[np.int32(slot), np.int32(1), sl] = lax.shift_right_logical(i10, two)
                idxb[np.int32(slot), np.int32(2), sl] = lax.shift_right_logical(i01, two)
                idxb[np.int32(slot), np.int32(3), sl] = lax.shift_right_logical(i11, two)
                dbuf[np.int32(slot), np.int32(0), sl] = d0
                dbuf[np.int32(slot), np.int32(1), sl] = d1
                qbuf[np.int32(slot), np.int32(0), sl] = (i00 & three) + (i00 & three)
                qbuf[np.int32(slot), np.int32(1), sl] = (i10 & three) + (i10 & three)
                qbuf[np.int32(slot), np.int32(2), sl] = (i01 & three) + (i01 & three)
                qbuf[np.int32(slot), np.int32(3), sl] = (i11 & three) + (i11 & three)
            return [pltpu.async_copy(
                trem_refs[li].at[idxb.at[np.int32(slot), np.int32(c)]],
                rows.at[np.int32(slot), pl.ds(np.int32(c * _C), _C)],
                sem) for c in range(4)]

        def combine_level(lvl, slot):
            rl = rows.at[np.int32(slot)]

            if True:

                @_loop32(_G, unroll=2)
                def dbody(g, lvl=lvl, rl=rl, slot=slot):
                    row = g * jnp.int32(16) + iota
                    sl = pl.ds(g * jnp.int32(16), 16)
                    d0 = dbuf[np.int32(slot), np.int32(0), sl]
                    d1 = dbuf[np.int32(slot), np.int32(1), sl]
                    q00 = qbuf[np.int32(slot), np.int32(0), sl]
                    q10 = qbuf[np.int32(slot), np.int32(1), sl]
                    q01 = qbuf[np.int32(slot), np.int32(2), sl]
                    q11 = qbuf[np.int32(slot), np.int32(3), sl]
                    w00, w10, w01, w11 = _weights(d0, d1)
                    one = jnp.int32(1)
                    r0 = row
                    r1 = jnp.int32(_C) + row
                    r2 = jnp.int32(2 * _C) + row
                    r3 = jnp.int32(3 * _C) + row
                    ft0 = (plsc.load_gather(rl, [r0, q00]) * w00
                           + plsc.load_gather(rl, [r1, q10]) * w10
                           + plsc.load_gather(rl, [r2, q01]) * w01
                           + plsc.load_gather(rl, [r3, q11]) * w11)
                    ft1 = (plsc.load_gather(rl, [r0, q00 + one]) * w00
                           + plsc.load_gather(rl, [r1, q10 + one]) * w10
                           + plsc.load_gather(rl, [r2, q01 + one]) * w01
                           + plsc.load_gather(rl, [r3, q11 + one]) * w11)
                    plsc.store_scatter(
                        outc, [prow + row, jnp.full((16,), 2 * lvl, jnp.int32)], ft0)
                    plsc.store_scatter(
                        outc, [prow + row, jnp.full((16,), 2 * lvl + 1, jnp.int32)], ft1)

        def local_levels(lo, hi):
            for lvl in range(lo, hi):

                @_loop32(_G, unroll=2)
                def cbody(g, lvl=lvl):
                    row, x0, x1 = point_group(g)
                    f0, f1, d0, d1 = _coords(x0, x1, _RESOLUTIONS[lvl])
                    i00, i10, i01, i11 = _corner_indices(lvl, f0, f1)
                    w00, w10, w01, w11 = _weights(d0, d1)
                    t = loc[lvl]
                    j00 = i00 + i00
                    j10 = i10 + i10
                    j01 = i01 + i01
                    j11 = i11 + i11
                    one = jnp.int32(1)
                    ft0 = (plsc.load_gather(t, [j00]) * w00
                           + plsc.load_gather(t, [j10]) * w10
                           + plsc.load_gather(t, [j01]) * w01
                           + plsc.load_gather(t, [j11]) * w11)
                    ft1 = (plsc.load_gather(t, [j00 + one]) * w00
                           + plsc.load_gather(t, [j10 + one]) * w10
                           + plsc.load_gather(t, [j01 + one]) * w01
                           + plsc.load_gather(t, [j11 + one]) * w11)
                    plsc.store_scatter(
                        outc, [prow + row, jnp.full((16,), 2 * lvl, jnp.int32)], ft0)
                    plsc.store_scatter(
                        outc, [prow + row, jnp.full((16,), 2 * lvl + 1, jnp.int32)], ft1)

        inflight = {}
        inflight[8] = fire_level(8, 0)
        inflight[9] = fire_level(9, 1)
        local_levels(0, _NUM_LOCAL)

        pbase = jnp.maximum(base - jnp.int32(_C), wbase)
        pltpu.make_async_copy(
            outc.at[pl.ds(jnp.int32(_C) - prow, _C)],
            out_ref.at[pl.ds(pbase, _C)], sem_o).wait()

        for lvl in range(_NUM_LOCAL, _NUM_LVL):
            slot = (lvl - _NUM_LOCAL) % _NSLOT
            for cp in inflight.pop(lvl):
                cp.wait()
            nxt = lvl + 2
            if nxt < _NUM_LVL:
                inflight[nxt] = fire_level(nxt, (nxt - _NUM_LOCAL) % _NSLOT)
            combine_level(lvl, slot)

        pltpu.async_copy(outc.at[pl.ds(prow, _C)],
                         out_ref.at[pl.ds(base, _C)], sem_o)

    lastp = np.int32((_NCHUNK - 1) & 1)
    pltpu.make_async_copy(
        outc.at[pl.ds(np.int32(lastp * _C), _C)],
        out_ref.at[pl.ds(lastbase, _C)], sem_o).wait()
    pltpu.make_async_copy(
        xf_ref.at[pl.ds(lastbase * jnp.int32(2), 2 * _C)],
        xc.at[pl.ds(np.int32(((_NCHUNK) & 1) * 2 * _C), 2 * _C)], sem_x).wait()


@jax.jit
def _grid_encode(xf, *tables):
    mesh = plsc.VectorSubcoreMesh(core_axis_name="c", subcore_axis_name="s",
                                  num_cores=2, num_subcores=16)
    scratch = [
        pltpu.VMEM((2 * 2 * _C,), jnp.float32),
    ] + [
        pltpu.VMEM((2 * _TABLE_SIZES[l],), jnp.float32) for l in range(_NUM_LOCAL)
    ] + [
        pltpu.VMEM((_NSLOT, 4, _C), jnp.int32),
        pltpu.VMEM((_NSLOT, 4 * _C, 8), jnp.float32),
        pltpu.VMEM((_NSLOT, 2, _C), jnp.float32),
        pltpu.VMEM((_NSLOT, 4, _C), jnp.int32),
        pltpu.VMEM((2 * _C, 2 * _NUM_LVL), jnp.float32),
        pltpu.SemaphoreType.DMA,
        pltpu.SemaphoreType.DMA,
        pltpu.SemaphoreType.DMA,
        pltpu.SemaphoreType.DMA,
    ]
    f = pl.kernel(
        _body,
        out_type=jax.ShapeDtypeStruct((_N_POINTS, 2 * _NUM_LVL), jnp.float32),
        mesh=mesh,
        scratch_types=scratch,
        compiler_params=pltpu.CompilerParams(needs_layout_passes=False,
                                             use_tc_tiling_on_sc=False),
    )
    return f(xf, *tables)


def kernel(x, tables):
    xf = jnp.reshape(x, (-1,))
    tin = [jnp.reshape(tables[l], (-1,)) for l in range(_NUM_LOCAL)]
    for l in range(_NUM_LOCAL, _NUM_LVL):
        t = jnp.reshape(tables[l], (-1,))
        pad = (-t.shape[0]) % 8
        if pad:
            t = jnp.concatenate([t, jnp.zeros((pad,), t.dtype)])
        tin.append(jnp.reshape(t, (-1, 8)))
    return _grid_encode(xf, *tin)


# speedup vs baseline: 1.1792x; 1.1792x over previous
import numpy as np
import jax
import jax.numpy as jnp
from jax import lax
from jax.experimental import pallas as pl
from jax.experimental.pallas import tpu as pltpu
from jax.experimental.pallas import tpu_sc as plsc

_NUM_LVL = 16
_MAX_RES = 2048
_MIN_RES = 16
_MAX_ENTRY = 2 ** 19
_MASK = _MAX_ENTRY - 1
_P0 = np.int32(3367900313 - (1 << 32))
_P1 = np.int32(2654435761 - (1 << 32))
_N_POINTS = 1048576

_factor_b = np.exp((np.log(_MAX_RES) - np.log(_MIN_RES)) / (_NUM_LVL - 1))
_RESOLUTIONS = [float(np.floor(_MIN_RES * _factor_b ** i)) for i in range(_NUM_LVL)]
_TABLE_SIZES = [int(min(r ** 2, _MAX_ENTRY)) for r in _RESOLUTIONS]

_NUM_LOCAL = 8
_NUM_REMOTE = _NUM_LVL - _NUM_LOCAL
_NSLOT = 4
_C = 128
_NW = 32
_PPW = _N_POINTS // _NW
_NCHUNK = _PPW // _C
_G = _C // 16


def _i32(v):
    return v if v.dtype == jnp.int32 else v.astype(jnp.int32)


def _loop32(n, unroll=None):
    def decorator(body):
        def fbody(i, c):
            body(c)
            return c + np.int32(1)
        lax.fori_loop(0, n, fbody, np.int32(0), unroll=unroll)
    return decorator


def _coords(x0, x1, res):
    rf = np.float32(res - 1.0)
    hi = np.float32(res - 1.0001)
    c0 = jnp.minimum(jnp.maximum(x0 * rf, np.float32(0.0)), hi)
    c1 = jnp.minimum(jnp.maximum(x1 * rf, np.float32(0.0)), hi)
    f0 = c0.astype(jnp.int32)
    f1 = c1.astype(jnp.int32)
    d0 = c0 - f0.astype(jnp.float32)
    d1 = c1 - f1.astype(jnp.float32)
    return f0, f1, d0, d1


def _corner_indices(lvl, f0, f1):
    if _TABLE_SIZES[lvl] == _MAX_ENTRY:
        a = f0 * _P0
        ap = a + _P0
        b = f1 * _P1
        bp = b + _P1
        i00 = (a ^ b) & _MASK
        i10 = (ap ^ b) & _MASK
        i01 = (a ^ bp) & _MASK
        i11 = (ap ^ bp) & _MASK
    else:
        r = np.int32(_RESOLUTIONS[lvl])
        i00 = f0 + f1 * r
        i10 = i00 + np.int32(1)
        i01 = i00 + r
        i11 = i01 + np.int32(1)
    return i00, i10, i01, i11


def _weights(d0, d1):
    mx = np.float32(1.0) - d0
    my = np.float32(1.0) - d1
    return mx * my, d0 * my, mx * d1, d0 * d1


def _body(*refs):
    xf_ref = refs[0]
    tloc_refs = refs[1:1 + _NUM_LOCAL]
    trem_refs = refs[1 + _NUM_LOCAL:1 + _NUM_LVL]
    out_ref = refs[1 + _NUM_LVL]
    s = 2 + _NUM_LVL
    xc = refs[s]
    loc = refs[s + 1:s + 1 + _NUM_LOCAL]
    idxb = refs[s + 1 + _NUM_LOCAL]
    rows = refs[s + 2 + _NUM_LOCAL]
    outc = refs[s + 3 + _NUM_LOCAL]
    sem = refs[s + 4 + _NUM_LOCAL]
    sem2 = refs[s + 5 + _NUM_LOCAL]
    sem_x = refs[s + 6 + _NUM_LOCAL]
    sem_o = refs[s + 7 + _NUM_LOCAL]

    iota = lax.iota(jnp.int32, 16)

    for l in range(_NUM_LOCAL):
        pltpu.async_copy(tloc_refs[l], loc[l], sem2).wait()

    wid = (_i32(lax.axis_index("s")) * jnp.int32(2)
           + _i32(lax.axis_index("c")))

    wbase = wid * jnp.int32(_PPW)
    lastbase = wbase + jnp.int32(_PPW - _C)

    pltpu.async_copy(xf_ref.at[pl.ds(wbase * jnp.int32(2), 2 * _C)],
                     xc.at[pl.ds(0, 2 * _C)], sem_x)
    pltpu.async_copy(outc.at[pl.ds(np.int32(_C), _C)],
                     out_ref.at[pl.ds(wbase, _C)], sem_o)

    @_loop32(_NCHUNK)
    def chunk_body(ci):
        base = wbase + ci * jnp.int32(_C)
        p = ci & jnp.int32(1)
        poff = p * jnp.int32(2 * _C)
        prow = p * jnp.int32(_C)

        pltpu.make_async_copy(
            xf_ref.at[pl.ds(base * jnp.int32(2), 2 * _C)],
            xc.at[pl.ds(poff, 2 * _C)], sem_x).wait()
        nbase = jnp.minimum(base + jnp.int32(_C), lastbase)
        pltpu.async_copy(
            xf_ref.at[pl.ds(nbase * jnp.int32(2), 2 * _C)],
            xc.at[pl.ds((jnp.int32(2 * _C)) - poff + jnp.int32(0) * nbase, 2 * _C)],
            sem_x)

        def point_group(g):
            row = g * jnp.int32(16) + iota
            rr = row + row + poff
            x0 = plsc.load_gather(xc, [rr])
            x1 = plsc.load_gather(xc, [rr + jnp.int32(1)])
            return row, x0, x1

        def fire_level(lvl, slot):
            li = lvl - _NUM_LOCAL

            @_loop32(_G)
            def bbody(g, lvl=lvl, slot=slot):
                row, x0, x1 = point_group(g)
                f0, f1, _d0, _d1 = _coords(x0, x1, _RESOLUTIONS[lvl])
                i00, i10, i01, i11 = _corner_indices(lvl, f0, f1)
                sl = pl.ds(g * jnp.int32(16), 16)
                two = jnp.int32(2)
                idxb[np.int32(slot), np.int32(0), sl] = lax.shift_right_logical(i00, two)
                idxb[np.int32(slot), np.int32(1), sl] = lax.shift_right_logical(i10, two)
                idxb[np.int32(slot), np.int32(2), sl] = lax.shift_right_logical(i01, two)
                idxb[np.int32(slot), np.int32(3), sl] = lax.shift_right_logical(i11, two)
            return [pltpu.async_copy(
                trem_refs[li].at[idxb.at[np.int32(slot), np.int32(c)]],
                rows.at[np.int32(slot), pl.ds(np.int32(c * _C), _C)],
                sem) for c in range(4)]

        def combine_level(lvl, slot):
            rl = rows.at[np.int32(slot)]

            if True:

                @_loop32(_G)
                def dbody(g, lvl=lvl, rl=rl):
                    row, x0, x1 = point_group(g)
                    f0, f1, d0, d1 = _coords(x0, x1, _RESOLUTIONS[lvl])
                    i00, i10, i01, i11 = _corner_indices(lvl, f0, f1)
                    w00, w10, w01, w11 = _weights(d0, d1)
                    three = jnp.int32(3)
                    q00 = (i00 & three) + (i00 & three)
                    q10 = (i10 & three) + (i10 & three)
                    q01 = (i01 & three) + (i01 & three)
                    q11 = (i11 & three) + (i11 & three)
                    one = jnp.int32(1)
                    r0 = row
                    r1 = jnp.int32(_C) + row
                    r2 = jnp.int32(2 * _C) + row
                    r3 = jnp.int32(3 * _C) + row
                    ft0 = (plsc.load_gather(rl, [r0, q00]) * w00
                           + plsc.load_gather(rl, [r1, q10]) * w10
                           + plsc.load_gather(rl, [r2, q01]) * w01
                           + plsc.load_gather(rl, [r3, q11]) * w11)
                    ft1 = (plsc.load_gather(rl, [r0, q00 + one]) * w00
                           + plsc.load_gather(rl, [r1, q10 + one]) * w10
                           + plsc.load_gather(rl, [r2, q01 + one]) * w01
                           + plsc.load_gather(rl, [r3, q11 + one]) * w11)
                    plsc.store_scatter(
                        outc, [prow + row, jnp.full((16,), 2 * lvl, jnp.int32)], ft0)
                    plsc.store_scatter(
                        outc, [prow + row, jnp.full((16,), 2 * lvl + 1, jnp.int32)], ft1)

        def local_levels(lo, hi):
            @_loop32(_G)
            def cbody(g):
                row, x0, x1 = point_group(g)
                orow = prow + row
                one = jnp.int32(1)
                for lvl in range(lo, hi):
                    f0, f1, d0, d1 = _coords(x0, x1, _RESOLUTIONS[lvl])
                    i00, i10, i01, i11 = _corner_indices(lvl, f0, f1)
                    w00, w10, w01, w11 = _weights(d0, d1)
                    t = loc[lvl]
                    j00 = i00 + i00
                    j10 = i10 + i10
                    j01 = i01 + i01
                    j11 = i11 + i11
                    ft0 = (plsc.load_gather(t, [j00]) * w00
                           + plsc.load_gather(t, [j10]) * w10
                           + plsc.load_gather(t, [j01]) * w01
                           + plsc.load_gather(t, [j11]) * w11)
                    ft1 = (plsc.load_gather(t, [j00 + one]) * w00
                           + plsc.load_gather(t, [j10 + one]) * w10
                           + plsc.load_gather(t, [j01 + one]) * w01
                           + plsc.load_gather(t, [j11 + one]) * w11)
                    plsc.store_scatter(
                        outc, [orow, jnp.full((16,), 2 * lvl, jnp.int32)], ft0)
                    plsc.store_scatter(
                        outc, [orow, jnp.full((16,), 2 * lvl + 1, jnp.int32)], ft1)

        inflight = {}
        inflight[8] = fire_level(8, 0)
        inflight[9] = fire_level(9, 1)
        inflight[10] = fire_level(10, 2)
        local_levels(0, _NUM_LOCAL)

        pbase = jnp.maximum(base - jnp.int32(_C), wbase)
        pltpu.make_async_copy(
            outc.at[pl.ds(jnp.int32(_C) - prow, _C)],
            out_ref.at[pl.ds(pbase, _C)], sem_o).wait()

        for lvl in range(_NUM_LOCAL, _NUM_LVL):
            slot = (lvl - _NUM_LOCAL) % _NSLOT
            for cp in inflight.pop(lvl):
                cp.wait()
            nxt = lvl + 3
            if nxt < _NUM_LVL:
                inflight[nxt] = fire_level(nxt, (nxt - _NUM_LOCAL) % _NSLOT)
            combine_level(lvl, slot)

        pltpu.async_copy(outc.at[pl.ds(prow, _C)],
                         out_ref.at[pl.ds(base, _C)], sem_o)

    lastp = np.int32((_NCHUNK - 1) & 1)
    pltpu.make_async_copy(
        outc.at[pl.ds(np.int32(lastp * _C), _C)],
        out_ref.at[pl.ds(lastbase, _C)], sem_o).wait()
    pltpu.make_async_copy(
        xf_ref.at[pl.ds(lastbase * jnp.int32(2), 2 * _C)],
        xc.at[pl.ds(np.int32(((_NCHUNK) & 1) * 2 * _C), 2 * _C)], sem_x).wait()


@jax.jit
def _grid_encode(xf, *tables):
    mesh = plsc.VectorSubcoreMesh(core_axis_name="c", subcore_axis_name="s",
                                  num_cores=2, num_subcores=16)
    scratch = [
        pltpu.VMEM((2 * 2 * _C,), jnp.float32),
    ] + [
        pltpu.VMEM((2 * _TABLE_SIZES[l],), jnp.float32) for l in range(_NUM_LOCAL)
    ] + [
        pltpu.VMEM((_NSLOT, 4, _C), jnp.int32),
        pltpu.VMEM((_NSLOT, 4 * _C, 8), jnp.float32),
        pltpu.VMEM((2 * _C, 2 * _NUM_LVL), jnp.float32),
        pltpu.SemaphoreType.DMA,
        pltpu.SemaphoreType.DMA,
        pltpu.SemaphoreType.DMA,
        pltpu.SemaphoreType.DMA,
    ]
    f = pl.kernel(
        _body,
        out_type=jax.ShapeDtypeStruct((_N_POINTS, 2 * _NUM_LVL), jnp.float32),
        mesh=mesh,
        scratch_types=scratch,
        compiler_params=pltpu.CompilerParams(needs_layout_passes=False,
                                             use_tc_tiling_on_sc=False),
    )
    return f(xf, *tables)


def kernel(x, tables):
    xf = jnp.reshape(x, (-1,))
    tin = [jnp.reshape(tables[l], (-1,)) for l in range(_NUM_LOCAL)]
    for l in range(_NUM_LOCAL, _NUM_LVL):
        t = jnp.reshape(tables[l], (-1,))
        pad = (-t.shape[0]) % 8
        if pad:
            t = jnp.concatenate([t, jnp.zeros((pad,), t.dtype)])
        tin.append(jnp.reshape(t, (-1, 8)))
    return _grid_encode(xf, *tin)
